# Initial kernel scaffold; baseline (speedup 1.0000x reference)
import jax
import jax.numpy as jnp
from jax.experimental import pallas as pl


def kernel(nf, edge_index, nef, W1_o2i, b1_o2i, W2_o2i, b2_o2i, W1_i2o, b1_i2o, W2_i2o, b2_i2o, W1_red, b1_red, W2_red, b2_red):
    raise NotImplementedError("write your pallas kernel here")


# speedup vs baseline: 5.4821x; 5.4821x over previous
import functools

import jax
import jax.numpy as jnp
from jax import lax
from jax.experimental import pallas as pl
from jax.experimental.pallas import tpu as pltpu
from jax.experimental.pallas import tpu_sc as plsc

N = 10000
E = 320000
IN_NF = 128
IN_EF = 16
OUT_NF = 128

NUM_CORES = 2
NUM_TILES = 16
NUM_WORKERS = NUM_CORES * NUM_TILES
CHUNK = 128
NCHUNK = 80
EDGES_PER_WORKER = CHUNK * NCHUNK
EP = EDGES_PER_WORKER * NUM_WORKERS
NP = 10112
ROWS_PER_TILE = NP // NUM_TILES


def _lane_perm(v, idx):
    dn = lax.GatherDimensionNumbers(offset_dims=(), collapsed_slice_dims=(0,),
                                    start_index_map=(0,))
    return lax.gather(v, idx[:, None], dn, slice_sizes=(1,),
                      mode=lax.GatherScatterMode.PROMISE_IN_BOUNDS)


def _edge_sc_kernel(ts_h, td_h, re_h, src_h, dst_h, zz_h, w0_h, b0_h,
                    sd_h, ss_h,
                    isv, idv, ga, gb, rb, pd, ps, w0s, b0s,
                    sdacc, ssacc, sem1, sem2):
    f32 = jnp.float32
    cid = lax.axis_index("c")
    sid = lax.axis_index("s")
    wid = sid * NUM_CORES + cid
    row0 = sid * ROWS_PER_TILE

    pltpu.sync_copy(zz_h.at[pl.ds(row0, ROWS_PER_TILE)],
                    sdacc.at[pl.ds(row0, ROWS_PER_TILE)])
    pltpu.sync_copy(zz_h.at[pl.ds(row0, ROWS_PER_TILE)],
                    ssacc.at[pl.ds(row0, ROWS_PER_TILE)])
    pltpu.sync_copy(w0_h, w0s)
    pltpu.sync_copy(b0_h, b0s)
    plsc.subcore_barrier()

    w0r = w0s[...]
    b0r = b0s[...]
    lane = lax.broadcasted_iota(jnp.int32, (16,), 0)
    one = jnp.full((16,), 1.0, f32)
    zero = jnp.full((16,), 0.0, f32)
    cntv = jnp.where(lane == 0, one, zero)
    px1 = jnp.bitwise_xor(lane, 1)
    px2 = jnp.bitwise_xor(lane, 2)
    px4 = jnp.bitwise_xor(lane, 4)
    px8 = jnp.bitwise_xor(lane, 8)
    ebase0 = wid * EDGES_PER_WORKER

    def chunk_body(c, carry):
        eb = pl.multiple_of(ebase0 + c * CHUNK, CHUNK)
        pltpu.sync_copy(src_h.at[pl.ds(eb, CHUNK)], isv)
        pltpu.sync_copy(dst_h.at[pl.ds(eb, CHUNK)], idv)
        pltpu.sync_copy(re_h.at[pl.ds(eb, CHUNK)], rb)
        cp1 = pltpu.async_copy(ts_h.at[isv], ga, sem1)
        cp2 = pltpu.async_copy(td_h.at[idv], gb, sem2)
        cp1.wait()
        cp2.wait()

        def edge_body(e, ec):
            a0 = ga[e, pl.ds(0, 16)]
            a1 = ga[e, pl.ds(16, 16)]
            c0 = gb[e, pl.ds(0, 16)]
            c1 = gb[e, pl.ds(16, 16)]
            r0 = rb[e, pl.ds(0, 16)]
            r1 = rb[e, pl.ds(16, 16)]
            h1 = a0 + c0 + r0
            g1 = jnp.where(h1 > 0, h1, 0.2 * h1)
            h2 = a1 + c1 + r1
            g2 = jnp.where(h2 > 0, h2, 0.2 * h2)
            sv = g2 * w0r
            sv = sv + _lane_perm(sv, px1)
            sv = sv + _lane_perm(sv, px2)
            sv = sv + _lane_perm(sv, px4)
            sv = sv + _lane_perm(sv, px8)
            kv = 1.0 / (1.0 + jnp.exp(-(sv + b0r)))
            u = kv * g2
            tail = jnp.where(lane == 0, kv, jnp.where(lane == 1, one, zero))
            pd[e, pl.ds(0, 16)] = g1
            pd[e, pl.ds(16, 16)] = cntv
            ps[e, pl.ds(0, 16)] = u
            ps[e, pl.ds(16, 16)] = tail
            return ec

        lax.fori_loop(0, CHUNK, edge_body, 0)
        pltpu.sync_copy(pd, sdacc.at[idv], add=True)
        pltpu.sync_copy(ps, ssacc.at[isv], add=True)
        return carry

    lax.fori_loop(0, NCHUNK, chunk_body, 0)
    plsc.subcore_barrier()
    pltpu.sync_copy(sdacc.at[pl.ds(row0, ROWS_PER_TILE)],
                    sd_h.at[cid, pl.ds(row0, ROWS_PER_TILE)])
    pltpu.sync_copy(ssacc.at[pl.ds(row0, ROWS_PER_TILE)],
                    ss_h.at[cid, pl.ds(row0, ROWS_PER_TILE)])


def _tables_body(nf_ref, ws_ref, wd_ref, ts_ref, td_ref):
    x = nf_ref[...]
    ts_ref[...] = jnp.dot(x, ws_ref[...], preferred_element_type=jnp.float32)
    td_ref[...] = jnp.dot(x, wd_ref[...], preferred_element_type=jnp.float32)


def _re_body(nef_ref, w_ref, b_ref, re_ref):
    re_ref[...] = (jnp.dot(nef_ref[...], w_ref[...],
                           preferred_element_type=jnp.float32) + b_ref[...])


def _fin_body(sd_ref, ss_ref, w2o_ref, b2o_ref, w2f_ref, b2f_ref,
              w1r_ref, b1r_ref, w2r_ref, b2r_ref, out_ref):
    f32 = jnp.float32
    sd = sd_ref[0] + sd_ref[1]
    ss = ss_ref[0] + ss_ref[1]
    s1 = sd[:, 0:16]
    cntd = sd[:, 16:17]
    new_nf = jnp.dot(s1, w2o_ref[...], preferred_element_type=f32) + cntd * b2o_ref[...]
    s2 = ss[:, 0:16]
    ks = ss[:, 16:17]
    cnts = ss[:, 17:18]
    nfo12 = jnp.dot(s2, w2f_ref[...], preferred_element_type=f32) + ks * b2f_ref[...]
    nfo2 = nfo12[:, 8:16] / jnp.maximum(cnts, 1.0)
    hin = jnp.concatenate([new_nf, nfo12[:, 0:8], nfo2], axis=1)
    h = jnp.dot(hin, w1r_ref[...], preferred_element_type=f32) + b1r_ref[...]
    h = jnp.where(h > 0, h, 0.2 * h)
    red = jnp.dot(h, w2r_ref[...], preferred_element_type=f32) + b2r_ref[...]
    out_ref[...] = jnp.where(cnts > 0, red, new_nf)


def kernel(nf, edge_index, nef,
           W1_o2i, b1_o2i, W2_o2i, b2_o2i,
           W1_i2o, b1_i2o, W2_i2o, b2_i2o,
           W1_red, b1_red, W2_red, b2_red):
    f32 = jnp.float32
    i32 = jnp.int32

    nf_p = jnp.concatenate([nf, jnp.zeros((NP - N, IN_NF), f32)], axis=0)
    ws = jnp.concatenate([W1_o2i[:IN_NF], W1_i2o[IN_NF:2 * IN_NF]], axis=1)
    wd = jnp.concatenate([W1_o2i[IN_NF:2 * IN_NF], W1_i2o[:IN_NF]], axis=1)
    wre = jnp.concatenate([W1_o2i[2 * IN_NF:], W1_i2o[2 * IN_NF:]], axis=1)
    bre = jnp.concatenate([b1_o2i, b1_i2o]).reshape(1, 32)
    nef_p = jnp.concatenate([nef, jnp.zeros((EP - E, IN_EF), f32)], axis=0)
    pad_idx = jnp.full((EP - E,), N, i32)
    src_p = jnp.concatenate([edge_index[0], pad_idx])
    dst_p = jnp.concatenate([edge_index[1], pad_idx])
    w0v = W2_i2o[:, 0]
    b0v = jnp.full((16,), 1.0, f32) * b2_i2o[0]
    zeros_acc = jnp.zeros((NP, 32), f32)

    ts, td = pl.pallas_call(
        _tables_body,
        out_shape=(jax.ShapeDtypeStruct((NP, 32), f32),
                   jax.ShapeDtypeStruct((NP, 32), f32)),
    )(nf_p, ws, wd)

    EBLK = 16384
    re = pl.pallas_call(
        _re_body,
        grid=(EP // EBLK,),
        in_specs=[pl.BlockSpec((EBLK, IN_EF), lambda i: (i, 0)),
                  pl.BlockSpec((IN_EF, 32), lambda i: (0, 0)),
                  pl.BlockSpec((1, 32), lambda i: (0, 0))],
        out_specs=pl.BlockSpec((EBLK, 32), lambda i: (i, 0)),
        out_shape=jax.ShapeDtypeStruct((EP, 32), f32),
    )(nef_p, wre, bre)

    mesh = plsc.VectorSubcoreMesh(core_axis_name="c", subcore_axis_name="s")
    edge_fn = functools.partial(
        pl.kernel,
        out_type=(jax.ShapeDtypeStruct((NUM_CORES, NP, 32), f32),
                  jax.ShapeDtypeStruct((NUM_CORES, NP, 32), f32)),
        mesh=mesh,
        scratch_types=[
            pltpu.VMEM((CHUNK,), i32),
            pltpu.VMEM((CHUNK,), i32),
            pltpu.VMEM((CHUNK, 32), f32),
            pltpu.VMEM((CHUNK, 32), f32),
            pltpu.VMEM((CHUNK, 32), f32),
            pltpu.VMEM((CHUNK, 32), f32),
            pltpu.VMEM((CHUNK, 32), f32),
            pltpu.VMEM((16,), f32),
            pltpu.VMEM((16,), f32),
            pltpu.VMEM_SHARED((NP, 32), f32),
            pltpu.VMEM_SHARED((NP, 32), f32),
            pltpu.SemaphoreType.DMA,
            pltpu.SemaphoreType.DMA,
        ],
        compiler_params=pltpu.CompilerParams(use_tc_tiling_on_sc=False),
    )(_edge_sc_kernel)
    sd_part, ss_part = edge_fn(ts, td, re, src_p, dst_p, zeros_acc, w0v, b0v)

    sdp = sd_part[:, :N]
    ssp = ss_part[:, :N]
    b2o = b2_o2i.reshape(1, OUT_NF)
    w2f = W2_i2o[:, 1:17]
    b2f = b2_i2o[1:17].reshape(1, 16)
    b1r = b1_red.reshape(1, 16)
    b2r = b2_red.reshape(1, OUT_NF)
    RBLK = 2000
    out = pl.pallas_call(
        _fin_body,
        grid=(N // RBLK,),
        in_specs=[pl.BlockSpec((NUM_CORES, RBLK, 32), lambda i: (0, i, 0)),
                  pl.BlockSpec((NUM_CORES, RBLK, 32), lambda i: (0, i, 0)),
                  pl.BlockSpec((16, OUT_NF), lambda i: (0, 0)),
                  pl.BlockSpec((1, OUT_NF), lambda i: (0, 0)),
                  pl.BlockSpec((16, 16), lambda i: (0, 0)),
                  pl.BlockSpec((1, 16), lambda i: (0, 0)),
                  pl.BlockSpec((144, 16), lambda i: (0, 0)),
                  pl.BlockSpec((1, 16), lambda i: (0, 0)),
                  pl.BlockSpec((16, OUT_NF), lambda i: (0, 0)),
                  pl.BlockSpec((1, OUT_NF), lambda i: (0, 0))],
        out_specs=pl.BlockSpec((RBLK, OUT_NF), lambda i: (i, 0)),
        out_shape=jax.ShapeDtypeStruct((N, OUT_NF), f32),
    )(sdp, ssp, W2_o2i, b2o, w2f, b2f, W1_red, b1r, W2_red, b2r)
    return out


# speedup vs baseline: 7.7073x; 1.4059x over previous
import functools

import jax
import jax.numpy as jnp
from jax import lax
from jax.experimental import pallas as pl
from jax.experimental.pallas import tpu as pltpu
from jax.experimental.pallas import tpu_sc as plsc

N = 10000
E = 320000
IN_NF = 128
IN_EF = 16
OUT_NF = 128

NUM_CORES = 2
NUM_TILES = 16
NUM_WORKERS = NUM_CORES * NUM_TILES
CHUNK = 128
NCHUNK = 80
EDGES_PER_WORKER = CHUNK * NCHUNK
EP = EDGES_PER_WORKER * NUM_WORKERS
NP = 10112
ROWS_PER_TILE = NP // NUM_TILES


def _lane_perm(v, idx):
    dn = lax.GatherDimensionNumbers(offset_dims=(), collapsed_slice_dims=(0,),
                                    start_index_map=(0,))
    return lax.gather(v, idx[:, None], dn, slice_sizes=(1,),
                      mode=lax.GatherScatterMode.PROMISE_IN_BOUNDS)


def _edge_sc_kernel(ts_h, td_h, re_h, src_h, dst_h, zz_h, w0_h, b0_h,
                    sd_h, ss_h,
                    isv2, idv2, ga2, gb2, rb2, pd, ps, w0s, b0s,
                    sdacc, ssacc,
                    sga0, sga1, sgb0, sgb1, sre0, sre1):
    f32 = jnp.float32
    cid = lax.axis_index("c")
    sid = lax.axis_index("s")
    wid = sid * NUM_CORES + cid
    row0 = sid * ROWS_PER_TILE
    crow0 = wid * NCHUNK

    pltpu.sync_copy(zz_h.at[pl.ds(row0, ROWS_PER_TILE)],
                    sdacc.at[pl.ds(row0, ROWS_PER_TILE)])
    pltpu.sync_copy(zz_h.at[pl.ds(row0, ROWS_PER_TILE)],
                    ssacc.at[pl.ds(row0, ROWS_PER_TILE)])
    pltpu.sync_copy(w0_h, w0s)
    pltpu.sync_copy(b0_h, b0s)
    pltpu.sync_copy(src_h.at[pl.ds(crow0, NCHUNK)], isv2)
    pltpu.sync_copy(dst_h.at[pl.ds(crow0, NCHUNK)], idv2)
    plsc.subcore_barrier()

    w0r = w0s[...]
    b0r = b0s[...]
    lane = lax.broadcasted_iota(jnp.int32, (16,), 0)
    one = jnp.full((16,), 1.0, f32)
    zero = jnp.full((16,), 0.0, f32)
    cntv = jnp.where(lane == 0, one, zero)
    px1 = jnp.bitwise_xor(lane, 1)
    px2 = jnp.bitwise_xor(lane, 2)
    px4 = jnp.bitwise_xor(lane, 4)
    px8 = jnp.bitwise_xor(lane, 8)
    sems = ((sga0, sgb0, sre0), (sga1, sgb1, sre1))
    bufs = ((ga2.at[0], gb2.at[0], rb2.at[0]), (ga2.at[1], gb2.at[1], rb2.at[1]))

    def _descs(c, b):
        eb = pl.multiple_of((crow0 + c) * CHUNK, CHUNK)
        ga_b, gb_b, rb_b = bufs[b]
        sga, sgb, sre = sems[b]
        return (pltpu.make_async_copy(ts_h.at[isv2.at[c]], ga_b, sga),
                pltpu.make_async_copy(td_h.at[idv2.at[c]], gb_b, sgb),
                pltpu.make_async_copy(re_h.at[pl.ds(eb, CHUNK)], rb_b, sre))

    def _fire(c, b):
        for d in _descs(c, b):
            d.start()

    def _wait(c, b):
        for d in _descs(c, b):
            d.wait()

    def _process(c, b):
        ga_b, gb_b, rb_b = bufs[b]

        def edge_body(e, ec):
            a0 = ga_b[e, pl.ds(0, 16)]
            a1 = ga_b[e, pl.ds(16, 16)]
            c0 = gb_b[e, pl.ds(0, 16)]
            c1 = gb_b[e, pl.ds(16, 16)]
            r0 = rb_b[e, pl.ds(0, 16)]
            r1 = rb_b[e, pl.ds(16, 16)]
            h1 = a0 + c0 + r0
            g1 = jnp.where(h1 > 0, h1, 0.2 * h1)
            h2 = a1 + c1 + r1
            g2 = jnp.where(h2 > 0, h2, 0.2 * h2)
            sv = g2 * w0r
            sv = sv + _lane_perm(sv, px1)
            sv = sv + _lane_perm(sv, px2)
            sv = sv + _lane_perm(sv, px4)
            sv = sv + _lane_perm(sv, px8)
            kv = 1.0 / (1.0 + jnp.exp(-(sv + b0r)))
            u = kv * g2
            tail = jnp.where(lane == 0, kv, jnp.where(lane == 1, one, zero))
            pd[e, pl.ds(0, 16)] = g1
            pd[e, pl.ds(16, 16)] = cntv
            ps[e, pl.ds(0, 16)] = u
            ps[e, pl.ds(16, 16)] = tail
            return ec

        lax.fori_loop(0, CHUNK, edge_body, 0)
        pltpu.sync_copy(pd, sdacc.at[idv2.at[c]], add=True)
        pltpu.sync_copy(ps, ssacc.at[isv2.at[c]], add=True)

    _fire(0, 0)

    def body(i, carry):
        c0 = 2 * i
        c1 = c0 + 1
        _fire(c1, 1)
        _wait(c0, 0)
        _process(c0, 0)

        @pl.when(i < NCHUNK // 2 - 1)
        def _():
            _fire(c0 + 2, 0)

        _wait(c1, 1)
        _process(c1, 1)
        return carry

    lax.fori_loop(0, NCHUNK // 2, body, 0)
    plsc.subcore_barrier()
    pltpu.sync_copy(sdacc.at[pl.ds(row0, ROWS_PER_TILE)],
                    sd_h.at[cid, pl.ds(row0, ROWS_PER_TILE)])
    pltpu.sync_copy(ssacc.at[pl.ds(row0, ROWS_PER_TILE)],
                    ss_h.at[cid, pl.ds(row0, ROWS_PER_TILE)])


def _tables_body(nf_ref, ws_ref, wd_ref, ts_ref, td_ref):
    x = nf_ref[...]
    ts_ref[...] = jnp.dot(x, ws_ref[...], preferred_element_type=jnp.float32)
    td_ref[...] = jnp.dot(x, wd_ref[...], preferred_element_type=jnp.float32)


def _re_body(nef_ref, w_ref, b_ref, re_ref):
    re_ref[...] = (jnp.dot(nef_ref[...], w_ref[...],
                           preferred_element_type=jnp.float32) + b_ref[...])


def _fin_body(sd_ref, ss_ref, w2o_ref, b2o_ref, w2f_ref, b2f_ref,
              w1r_ref, b1r_ref, w2r_ref, b2r_ref, out_ref):
    f32 = jnp.float32
    sd = sd_ref[0] + sd_ref[1]
    ss = ss_ref[0] + ss_ref[1]
    s1 = sd[:, 0:16]
    cntd = sd[:, 16:17]
    new_nf = jnp.dot(s1, w2o_ref[...], preferred_element_type=f32) + cntd * b2o_ref[...]
    s2 = ss[:, 0:16]
    ks = ss[:, 16:17]
    cnts = ss[:, 17:18]
    nfo12 = jnp.dot(s2, w2f_ref[...], preferred_element_type=f32) + ks * b2f_ref[...]
    nfo2 = nfo12[:, 8:16] / jnp.maximum(cnts, 1.0)
    hin = jnp.concatenate([new_nf, nfo12[:, 0:8], nfo2], axis=1)
    h = jnp.dot(hin, w1r_ref[...], preferred_element_type=f32) + b1r_ref[...]
    h = jnp.where(h > 0, h, 0.2 * h)
    red = jnp.dot(h, w2r_ref[...], preferred_element_type=f32) + b2r_ref[...]
    out_ref[...] = jnp.where(cnts > 0, red, new_nf)


def kernel(nf, edge_index, nef,
           W1_o2i, b1_o2i, W2_o2i, b2_o2i,
           W1_i2o, b1_i2o, W2_i2o, b2_i2o,
           W1_red, b1_red, W2_red, b2_red):
    f32 = jnp.float32
    i32 = jnp.int32

    nf_p = jnp.concatenate([nf, jnp.zeros((NP - N, IN_NF), f32)], axis=0)
    ws = jnp.concatenate([W1_o2i[:IN_NF], W1_i2o[IN_NF:2 * IN_NF]], axis=1)
    wd = jnp.concatenate([W1_o2i[IN_NF:2 * IN_NF], W1_i2o[:IN_NF]], axis=1)
    wre = jnp.concatenate([W1_o2i[2 * IN_NF:], W1_i2o[2 * IN_NF:]], axis=1)
    bre = jnp.concatenate([b1_o2i, b1_i2o]).reshape(1, 32)
    nef_p = jnp.concatenate([nef, jnp.zeros((EP - E, IN_EF), f32)], axis=0)
    pad_idx = jnp.full((EP - E,), N, i32)
    src_p = jnp.concatenate([edge_index[0], pad_idx]).reshape(EP // CHUNK, CHUNK)
    dst_p = jnp.concatenate([edge_index[1], pad_idx]).reshape(EP // CHUNK, CHUNK)
    w0v = W2_i2o[:, 0]
    b0v = jnp.full((16,), 1.0, f32) * b2_i2o[0]
    zeros_acc = jnp.zeros((NP, 32), f32)

    ts, td = pl.pallas_call(
        _tables_body,
        out_shape=(jax.ShapeDtypeStruct((NP, 32), f32),
                   jax.ShapeDtypeStruct((NP, 32), f32)),
    )(nf_p, ws, wd)

    EBLK = 16384
    re = pl.pallas_call(
        _re_body,
        grid=(EP // EBLK,),
        in_specs=[pl.BlockSpec((EBLK, IN_EF), lambda i: (i, 0)),
                  pl.BlockSpec((IN_EF, 32), lambda i: (0, 0)),
                  pl.BlockSpec((1, 32), lambda i: (0, 0))],
        out_specs=pl.BlockSpec((EBLK, 32), lambda i: (i, 0)),
        out_shape=jax.ShapeDtypeStruct((EP, 32), f32),
    )(nef_p, wre, bre)

    mesh = plsc.VectorSubcoreMesh(core_axis_name="c", subcore_axis_name="s")
    edge_fn = functools.partial(
        pl.kernel,
        out_type=(jax.ShapeDtypeStruct((NUM_CORES, NP, 32), f32),
                  jax.ShapeDtypeStruct((NUM_CORES, NP, 32), f32)),
        mesh=mesh,
        scratch_types=[
            pltpu.VMEM((NCHUNK, CHUNK), i32),
            pltpu.VMEM((NCHUNK, CHUNK), i32),
            pltpu.VMEM((2, CHUNK, 32), f32),
            pltpu.VMEM((2, CHUNK, 32), f32),
            pltpu.VMEM((2, CHUNK, 32), f32),
            pltpu.VMEM((CHUNK, 32), f32),
            pltpu.VMEM((CHUNK, 32), f32),
            pltpu.VMEM((16,), f32),
            pltpu.VMEM((16,), f32),
            pltpu.VMEM_SHARED((NP, 32), f32),
            pltpu.VMEM_SHARED((NP, 32), f32),
            pltpu.SemaphoreType.DMA,
            pltpu.SemaphoreType.DMA,
            pltpu.SemaphoreType.DMA,
            pltpu.SemaphoreType.DMA,
            pltpu.SemaphoreType.DMA,
            pltpu.SemaphoreType.DMA,
        ],
        compiler_params=pltpu.CompilerParams(use_tc_tiling_on_sc=False),
    )(_edge_sc_kernel)
    sd_part, ss_part = edge_fn(ts, td, re, src_p, dst_p, zeros_acc, w0v, b0v)

    b2o = b2_o2i.reshape(1, OUT_NF)
    w2f = W2_i2o[:, 1:17]
    b2f = b2_i2o[1:17].reshape(1, 16)
    b1r = b1_red.reshape(1, 16)
    b2r = b2_red.reshape(1, OUT_NF)
    RBLK = 2000
    out = pl.pallas_call(
        _fin_body,
        grid=(N // RBLK,),
        in_specs=[pl.BlockSpec((NUM_CORES, RBLK, 32), lambda i: (0, i, 0)),
                  pl.BlockSpec((NUM_CORES, RBLK, 32), lambda i: (0, i, 0)),
                  pl.BlockSpec((16, OUT_NF), lambda i: (0, 0)),
                  pl.BlockSpec((1, OUT_NF), lambda i: (0, 0)),
                  pl.BlockSpec((16, 16), lambda i: (0, 0)),
                  pl.BlockSpec((1, 16), lambda i: (0, 0)),
                  pl.BlockSpec((144, 16), lambda i: (0, 0)),
                  pl.BlockSpec((1, 16), lambda i: (0, 0)),
                  pl.BlockSpec((16, OUT_NF), lambda i: (0, 0)),
                  pl.BlockSpec((1, OUT_NF), lambda i: (0, 0))],
        out_specs=pl.BlockSpec((RBLK, OUT_NF), lambda i: (i, 0)),
        out_shape=jax.ShapeDtypeStruct((N, OUT_NF), f32),
    )(sd_part, ss_part, W2_o2i, b2o, w2f, b2f, W1_red, b1r, W2_red, b2r)
    return out


# speedup vs baseline: 9.0885x; 1.1792x over previous
import functools

import jax
import jax.numpy as jnp
from jax import lax
from jax.experimental import pallas as pl
from jax.experimental.pallas import tpu as pltpu
from jax.experimental.pallas import tpu_sc as plsc

N = 10000
E = 320000
IN_NF = 128
IN_EF = 16
OUT_NF = 128

NUM_CORES = 2
NUM_TILES = 16
NUM_WORKERS = NUM_CORES * NUM_TILES
CHUNK = 128
TOTAL_ROWS = E // CHUNK
BASE_ROWS = TOTAL_ROWS // NUM_WORKERS
LEFTOVER = TOTAL_ROWS - BASE_ROWS * NUM_WORKERS
NP = 10112
ROWS_PER_TILE = NP // NUM_TILES


def _lane_perm(v, idx):
    dn = lax.GatherDimensionNumbers(offset_dims=(), collapsed_slice_dims=(0,),
                                    start_index_map=(0,))
    return lax.gather(v, idx[:, None], dn, slice_sizes=(1,),
                      mode=lax.GatherScatterMode.PROMISE_IN_BOUNDS)


def _edge_sc_kernel(ts_h, td_h, re_h, src_h, dst_h, zz_h, w0_h, b0_h,
                    sd_h, ss_h,
                    isv2, idv2, isx, idx1, ga2, gb2, rb2, pd, ps, w0s, b0s,
                    sdacc, ssacc,
                    sga0, sga1, sgb0, sgb1, sre0, sre1):
    f32 = jnp.float32
    cid = lax.axis_index("c")
    sid = lax.axis_index("s")
    wid = sid * NUM_CORES + cid
    row0 = sid * ROWS_PER_TILE
    crow0 = wid * BASE_ROWS

    pltpu.sync_copy(zz_h.at[pl.ds(row0, ROWS_PER_TILE)],
                    sdacc.at[pl.ds(row0, ROWS_PER_TILE)])
    pltpu.sync_copy(zz_h.at[pl.ds(row0, ROWS_PER_TILE)],
                    ssacc.at[pl.ds(row0, ROWS_PER_TILE)])
    pltpu.sync_copy(w0_h, w0s)
    pltpu.sync_copy(b0_h, b0s)
    pltpu.sync_copy(src_h.at[pl.ds(crow0, BASE_ROWS)], isv2)
    pltpu.sync_copy(dst_h.at[pl.ds(crow0, BASE_ROWS)], idv2)
    plsc.subcore_barrier()

    w0r = w0s[...]
    b0r = b0s[...]
    lane = lax.broadcasted_iota(jnp.int32, (16,), 0)
    one = jnp.full((16,), 1.0, f32)
    zero = jnp.full((16,), 0.0, f32)
    cntv = jnp.where(lane == 0, one, zero)
    px1 = jnp.bitwise_xor(lane, 1)
    px2 = jnp.bitwise_xor(lane, 2)
    px4 = jnp.bitwise_xor(lane, 4)
    px8 = jnp.bitwise_xor(lane, 8)
    sems = ((sga0, sgb0, sre0), (sga1, sgb1, sre1))
    bufs = ((ga2.at[0], gb2.at[0], rb2.at[0]), (ga2.at[1], gb2.at[1], rb2.at[1]))

    def _descs(b, is_row, id_row, eb):
        ga_b, gb_b, rb_b = bufs[b]
        sga, sgb, sre = sems[b]
        return (pltpu.make_async_copy(ts_h.at[is_row], ga_b, sga),
                pltpu.make_async_copy(td_h.at[id_row], gb_b, sgb),
                pltpu.make_async_copy(re_h.at[pl.ds(eb, CHUNK)], rb_b, sre))

    def _main_descs(c, b):
        eb = pl.multiple_of((crow0 + c) * CHUNK, CHUNK)
        return _descs(b, isv2.at[c], idv2.at[c], eb)

    def _fire(c, b):
        for d in _main_descs(c, b):
            d.start()

    def _wait(c, b):
        for d in _main_descs(c, b):
            d.wait()

    def _compute(b):
        ga_b, gb_b, rb_b = bufs[b]

        def edge_body(e, ec):
            a0 = ga_b[e, pl.ds(0, 16)]
            a1 = ga_b[e, pl.ds(16, 16)]
            c0 = gb_b[e, pl.ds(0, 16)]
            c1 = gb_b[e, pl.ds(16, 16)]
            r0 = rb_b[e, pl.ds(0, 16)]
            r1 = rb_b[e, pl.ds(16, 16)]
            h1 = a0 + c0 + r0
            g1 = jnp.where(h1 > 0, h1, 0.2 * h1)
            h2 = a1 + c1 + r1
            g2 = jnp.where(h2 > 0, h2, 0.2 * h2)
            sv = g2 * w0r
            sv = sv + _lane_perm(sv, px1)
            sv = sv + _lane_perm(sv, px2)
            sv = sv + _lane_perm(sv, px4)
            sv = sv + _lane_perm(sv, px8)
            kv = 1.0 / (1.0 + jnp.exp(-(sv + b0r)))
            u = kv * g2
            tail = jnp.where(lane == 0, kv, jnp.where(lane == 1, one, zero))
            pd[e, pl.ds(0, 16)] = g1
            pd[e, pl.ds(16, 16)] = cntv
            ps[e, pl.ds(0, 16)] = u
            ps[e, pl.ds(16, 16)] = tail
            return ec

        lax.fori_loop(0, CHUNK, edge_body, 0)

    def _scatter(is_row, id_row):
        pltpu.sync_copy(pd, sdacc.at[id_row], add=True)
        pltpu.sync_copy(ps, ssacc.at[is_row], add=True)

    _fire(0, 0)

    def body(i, carry):
        c0 = 2 * i
        c1 = c0 + 1
        _fire(c1, 1)
        _wait(c0, 0)
        _compute(0)
        _scatter(isv2.at[c0], idv2.at[c0])

        @pl.when(i < BASE_ROWS // 2 - 1)
        def _():
            _fire(c0 + 2, 0)

        _wait(c1, 1)
        _compute(1)
        _scatter(isv2.at[c1], idv2.at[c1])
        return carry

    lax.fori_loop(0, BASE_ROWS // 2, body, 0)

    @pl.when(wid < LEFTOVER)
    def _():
        erow = TOTAL_ROWS - LEFTOVER + wid
        pltpu.sync_copy(src_h.at[pl.ds(erow, 1)], isx)
        pltpu.sync_copy(dst_h.at[pl.ds(erow, 1)], idx1)
        eb = erow * CHUNK
        for d in _descs(0, isx.at[0], idx1.at[0], eb):
            d.start()
        for d in _descs(0, isx.at[0], idx1.at[0], eb):
            d.wait()
        _compute(0)
        _scatter(isx.at[0], idx1.at[0])

    plsc.subcore_barrier()
    pltpu.sync_copy(sdacc.at[pl.ds(row0, ROWS_PER_TILE)],
                    sd_h.at[cid, pl.ds(row0, ROWS_PER_TILE)])
    pltpu.sync_copy(ssacc.at[pl.ds(row0, ROWS_PER_TILE)],
                    ss_h.at[cid, pl.ds(row0, ROWS_PER_TILE)])


def _pre_body(nef_ref, wre_ref, bre_ref, nf_ref, ws_ref, wd_ref,
              re_ref, ts_ref, td_ref):
    @pl.when(pl.program_id(0) == 0)
    def _():
        x = nf_ref[...]
        ts_ref[...] = jnp.dot(x, ws_ref[...], preferred_element_type=jnp.float32)
        td_ref[...] = jnp.dot(x, wd_ref[...], preferred_element_type=jnp.float32)

    re_ref[...] = (jnp.dot(nef_ref[...], wre_ref[...],
                           preferred_element_type=jnp.float32) + bre_ref[...])


def _fin_body(sd_ref, ss_ref, w2o_ref, b2o_ref, w2f_ref, b2f_ref,
              w1r_ref, b1r_ref, w2r_ref, b2r_ref, out_ref):
    f32 = jnp.float32
    sd = sd_ref[0] + sd_ref[1]
    ss = ss_ref[0] + ss_ref[1]
    s1 = sd[:, 0:16]
    cntd = sd[:, 16:17]
    new_nf = jnp.dot(s1, w2o_ref[...], preferred_element_type=f32) + cntd * b2o_ref[...]
    s2 = ss[:, 0:16]
    ks = ss[:, 16:17]
    cnts = ss[:, 17:18]
    nfo12 = jnp.dot(s2, w2f_ref[...], preferred_element_type=f32) + ks * b2f_ref[...]
    nfo2 = nfo12[:, 8:16] / jnp.maximum(cnts, 1.0)
    hin = jnp.concatenate([new_nf, nfo12[:, 0:8], nfo2], axis=1)
    h = jnp.dot(hin, w1r_ref[...], preferred_element_type=f32) + b1r_ref[...]
    h = jnp.where(h > 0, h, 0.2 * h)
    red = jnp.dot(h, w2r_ref[...], preferred_element_type=f32) + b2r_ref[...]
    out_ref[...] = jnp.where(cnts > 0, red, new_nf)


def kernel(nf, edge_index, nef,
           W1_o2i, b1_o2i, W2_o2i, b2_o2i,
           W1_i2o, b1_i2o, W2_i2o, b2_i2o,
           W1_red, b1_red, W2_red, b2_red):
    f32 = jnp.float32
    i32 = jnp.int32

    ws = jnp.concatenate([W1_o2i[:IN_NF], W1_i2o[IN_NF:2 * IN_NF]], axis=1)
    wd = jnp.concatenate([W1_o2i[IN_NF:2 * IN_NF], W1_i2o[:IN_NF]], axis=1)
    wre = jnp.concatenate([W1_o2i[2 * IN_NF:], W1_i2o[2 * IN_NF:]], axis=1)
    bre = jnp.concatenate([b1_o2i, b1_i2o]).reshape(1, 32)
    src_p = edge_index[0].reshape(TOTAL_ROWS, CHUNK)
    dst_p = edge_index[1].reshape(TOTAL_ROWS, CHUNK)
    w0v = W2_i2o[:, 0]
    b0v = jnp.full((16,), 1.0, f32) * b2_i2o[0]
    zeros_acc = jnp.zeros((NP, 32), f32)

    EBLK = 16000
    re, ts, td = pl.pallas_call(
        _pre_body,
        grid=(E // EBLK,),
        in_specs=[pl.BlockSpec((EBLK, IN_EF), lambda i: (i, 0)),
                  pl.BlockSpec((IN_EF, 32), lambda i: (0, 0)),
                  pl.BlockSpec((1, 32), lambda i: (0, 0)),
                  pl.BlockSpec((N, IN_NF), lambda i: (0, 0)),
                  pl.BlockSpec((IN_NF, 32), lambda i: (0, 0)),
                  pl.BlockSpec((IN_NF, 32), lambda i: (0, 0))],
        out_specs=(pl.BlockSpec((EBLK, 32), lambda i: (i, 0)),
                   pl.BlockSpec((N, 32), lambda i: (0, 0)),
                   pl.BlockSpec((N, 32), lambda i: (0, 0))),
        out_shape=(jax.ShapeDtypeStruct((E, 32), f32),
                   jax.ShapeDtypeStruct((N, 32), f32),
                   jax.ShapeDtypeStruct((N, 32), f32)),
    )(nef, wre, bre, nf, ws, wd)

    mesh = plsc.VectorSubcoreMesh(core_axis_name="c", subcore_axis_name="s")
    edge_fn = functools.partial(
        pl.kernel,
        out_type=(jax.ShapeDtypeStruct((NUM_CORES, NP, 32), f32),
                  jax.ShapeDtypeStruct((NUM_CORES, NP, 32), f32)),
        mesh=mesh,
        scratch_types=[
            pltpu.VMEM((BASE_ROWS, CHUNK), i32),
            pltpu.VMEM((BASE_ROWS, CHUNK), i32),
            pltpu.VMEM((1, CHUNK), i32),
            pltpu.VMEM((1, CHUNK), i32),
            pltpu.VMEM((2, CHUNK, 32), f32),
            pltpu.VMEM((2, CHUNK, 32), f32),
            pltpu.VMEM((2, CHUNK, 32), f32),
            pltpu.VMEM((CHUNK, 32), f32),
            pltpu.VMEM((CHUNK, 32), f32),
            pltpu.VMEM((16,), f32),
            pltpu.VMEM((16,), f32),
            pltpu.VMEM_SHARED((NP, 32), f32),
            pltpu.VMEM_SHARED((NP, 32), f32),
            pltpu.SemaphoreType.DMA,
            pltpu.SemaphoreType.DMA,
            pltpu.SemaphoreType.DMA,
            pltpu.SemaphoreType.DMA,
            pltpu.SemaphoreType.DMA,
            pltpu.SemaphoreType.DMA,
        ],
        compiler_params=pltpu.CompilerParams(use_tc_tiling_on_sc=False),
    )(_edge_sc_kernel)
    sd_part, ss_part = edge_fn(ts, td, re, src_p, dst_p, zeros_acc, w0v, b0v)

    b2o = b2_o2i.reshape(1, OUT_NF)
    w2f = W2_i2o[:, 1:17]
    b2f = b2_i2o[1:17].reshape(1, 16)
    b1r = b1_red.reshape(1, 16)
    b2r = b2_red.reshape(1, OUT_NF)
    RBLK = 2000
    out = pl.pallas_call(
        _fin_body,
        grid=(N // RBLK,),
        in_specs=[pl.BlockSpec((NUM_CORES, RBLK, 32), lambda i: (0, i, 0)),
                  pl.BlockSpec((NUM_CORES, RBLK, 32), lambda i: (0, i, 0)),
                  pl.BlockSpec((16, OUT_NF), lambda i: (0, 0)),
                  pl.BlockSpec((1, OUT_NF), lambda i: (0, 0)),
                  pl.BlockSpec((16, 16), lambda i: (0, 0)),
                  pl.BlockSpec((1, 16), lambda i: (0, 0)),
                  pl.BlockSpec((144, 16), lambda i: (0, 0)),
                  pl.BlockSpec((1, 16), lambda i: (0, 0)),
                  pl.BlockSpec((16, OUT_NF), lambda i: (0, 0)),
                  pl.BlockSpec((1, OUT_NF), lambda i: (0, 0))],
        out_specs=pl.BlockSpec((RBLK, OUT_NF), lambda i: (i, 0)),
        out_shape=jax.ShapeDtypeStruct((N, OUT_NF), f32),
    )(sd_part, ss_part, W2_o2i, b2o, w2f, b2f, W1_red, b1r, W2_red, b2r)
    return out
